# baseline (speedup 1.0000x reference)
import jax
import jax.numpy as jnp
from jax.experimental import pallas as pl
from jax.experimental.pallas import tpu as pltpu

_VMEM_LIMIT = 64 * 1024 * 1024


def _params(dims):
    return pltpu.CompilerParams(dimension_semantics=dims,
                                vmem_limit_bytes=_VMEM_LIMIT)


def _conv3x3(xs, w, b, *, rb, pool=False, head=None):
    B, H, W, _ = xs[0].shape
    cins = tuple(int(x.shape[-1]) for x in xs)
    ctot = sum(cins)
    Cout = int(w.shape[-1])
    n = len(xs)
    RB = min(rb, H)
    assert H % RB == 0 and (not pool or RB % 2 == 0)
    NB = H // RB
    CT = Cout if Cout <= 256 else 256
    NC = Cout // CT
    Wp = ((W + 2 + 15) // 16) * 16
    PW = Wp - W
    M = RB * Wp
    FLAT = 16 + (RB + 2) * Wp + 16
    CP = int(head[0].shape[-1]) if head is not None else 0

    def _body(*refs):
        x_refs = refs[:3 * n]
        w_ref = refs[3 * n]
        b_ref = refs[3 * n + 1]
        pos = 3 * n + 2
        if head is not None:
            wo_ref, bo_ref = refs[pos], refs[pos + 1]
            pos += 2
        o_ref = refs[pos]
        p_ref = refs[pos + 1] if pool else None
        xf, xl, xr = refs[-3], refs[-2], refs[-1]

        i = pl.program_id(1)
        first = i == 0
        last = i == NB - 1

        @pl.when(pl.program_id(2) == 0)
        def _stage():
            off = 0
            for j in range(n):
                cin = cins[j]
                top_ref, mid_ref, bot_ref = x_refs[3 * j:3 * j + 3]
                lanes = slice(off, off + cin)
                zrow = jnp.zeros((W, cin), jnp.bfloat16)
                zpad = jnp.zeros((PW, cin), jnp.bfloat16)
                xf[pl.ds(0, 16), lanes] = jnp.zeros((16, cin), jnp.bfloat16)
                xf[pl.ds(16, W), lanes] = jnp.where(first, zrow, top_ref[0, 0])
                xf[pl.ds(16 + W, PW), lanes] = zpad
                for r in range(RB):
                    base = 16 + (r + 1) * Wp
                    xf[pl.ds(base, W), lanes] = mid_ref[0, r]
                    xf[pl.ds(base + W, PW), lanes] = zpad
                base = 16 + (RB + 1) * Wp
                xf[pl.ds(base, W), lanes] = jnp.where(last, zrow, bot_ref[0, 0])
                xf[pl.ds(base + W, PW), lanes] = zpad
                xf[pl.ds(16 + (RB + 2) * Wp, 16), lanes] = \
                    jnp.zeros((16, cin), jnp.bfloat16)
                off += cin
            xr[pl.ds(1, FLAT - 1), :] = xf[pl.ds(0, FLAT - 1), :]
            xl[pl.ds(0, FLAT - 1), :] = xf[pl.ds(1, FLAT - 1), :]

        srcs = (xr, xf, xl)
        acc = jnp.zeros((M, CT), jnp.float32)
        for dy in range(3):
            for dx in range(3):
                lhs = srcs[dx][pl.ds(16 + dy * Wp, M), :]
                acc = acc + jnp.dot(lhs, w_ref[dy * 3 + dx],
                                    preferred_element_type=jnp.float32)
        acc = jnp.maximum(acc + b_ref[...], 0.0)
        y = acc.reshape(RB, Wp, CT)[:, :W, :]
        if head is not None:
            yb = y.astype(jnp.bfloat16).reshape(RB * W, CT)
            lg = jnp.dot(yb, wo_ref[...],
                         preferred_element_type=jnp.float32) + bo_ref[...]
            o_ref[0] = lg.reshape(RB, W, CP)
        else:
            o_ref[0] = y.astype(jnp.bfloat16)
            if pool:
                t = jnp.max(y.reshape(RB // 2, 2, W, CT), axis=1)
                t = jnp.max(t.reshape(RB // 2, W // 2, 2, CT), axis=2)
                p_ref[0] = t.astype(jnp.bfloat16)

    in_specs, inputs = [], []
    for x, cin in zip(xs, cins):
        in_specs += [
            pl.BlockSpec((1, 1, W, cin),
                         lambda bb, ii, cc: (bb, jnp.maximum(ii * RB - 1, 0), 0, 0)),
            pl.BlockSpec((1, RB, W, cin),
                         lambda bb, ii, cc: (bb, ii, 0, 0)),
            pl.BlockSpec((1, 1, W, cin),
                         lambda bb, ii, cc: (bb, jnp.minimum(ii * RB + RB, H - 1), 0, 0)),
        ]
        inputs += [x, x, x]
    in_specs.append(pl.BlockSpec((9, ctot, CT), lambda bb, ii, cc: (0, 0, cc)))
    inputs.append(w)
    in_specs.append(pl.BlockSpec((1, CT), lambda bb, ii, cc: (0, cc)))
    inputs.append(b)
    if head is not None:
        in_specs.append(pl.BlockSpec((CT, CP), lambda bb, ii, cc: (0, 0)))
        inputs.append(head[0])
        in_specs.append(pl.BlockSpec((1, CP), lambda bb, ii, cc: (0, 0)))
        inputs.append(head[1])

    if head is not None:
        out_shape = jax.ShapeDtypeStruct((B, H, W, CP), jnp.float32)
        out_specs = pl.BlockSpec((1, RB, W, CP), lambda bb, ii, cc: (bb, ii, 0, 0))
    elif pool:
        out_shape = (jax.ShapeDtypeStruct((B, H, W, Cout), jnp.bfloat16),
                     jax.ShapeDtypeStruct((B, H // 2, W // 2, Cout), jnp.bfloat16))
        out_specs = (pl.BlockSpec((1, RB, W, CT), lambda bb, ii, cc: (bb, ii, 0, cc)),
                     pl.BlockSpec((1, RB // 2, W // 2, CT),
                                  lambda bb, ii, cc: (bb, ii, 0, cc)))
    else:
        out_shape = jax.ShapeDtypeStruct((B, H, W, Cout), jnp.bfloat16)
        out_specs = pl.BlockSpec((1, RB, W, CT), lambda bb, ii, cc: (bb, ii, 0, cc))

    return pl.pallas_call(
        _body,
        out_shape=out_shape,
        grid=(B, NB, NC),
        in_specs=in_specs,
        out_specs=out_specs,
        scratch_shapes=[pltpu.VMEM((FLAT, ctot), jnp.bfloat16),
                        pltpu.VMEM((FLAT, ctot), jnp.bfloat16),
                        pltpu.VMEM((FLAT, ctot), jnp.bfloat16)],
        compiler_params=_params(("parallel", "parallel", "arbitrary")),
    )(*inputs)


def _stem(xcol, w, b, *, rb):
    B, H, W, K = xcol.shape
    Cout = int(w.shape[-1])
    RB = min(rb, H)
    NB = H // RB

    def _body(x_ref, w_ref, b_ref, o_ref, p_ref):
        acc = jnp.dot(x_ref[0].reshape(RB * W, K), w_ref[...],
                      preferred_element_type=jnp.float32)
        y = jnp.maximum(acc + b_ref[...], 0.0).reshape(RB, W, Cout)
        o_ref[0] = y.astype(jnp.bfloat16)
        t = jnp.max(y.reshape(RB // 2, 2, W, Cout), axis=1)
        t = jnp.max(t.reshape(RB // 2, W // 2, 2, Cout), axis=2)
        p_ref[0] = t.astype(jnp.bfloat16)

    return pl.pallas_call(
        _body,
        out_shape=(jax.ShapeDtypeStruct((B, H, W, Cout), jnp.bfloat16),
                   jax.ShapeDtypeStruct((B, H // 2, W // 2, Cout), jnp.bfloat16)),
        grid=(B, NB),
        in_specs=[
            pl.BlockSpec((1, RB, W, K), lambda bb, ii: (bb, ii, 0, 0)),
            pl.BlockSpec((K, Cout), lambda bb, ii: (0, 0)),
            pl.BlockSpec((1, Cout), lambda bb, ii: (0, 0)),
        ],
        out_specs=(pl.BlockSpec((1, RB, W, Cout), lambda bb, ii: (bb, ii, 0, 0)),
                   pl.BlockSpec((1, RB // 2, W // 2, Cout),
                                lambda bb, ii: (bb, ii, 0, 0))),
        compiler_params=_params(("parallel", "parallel")),
    )(xcol, w, b)


def _convT(x, w_cat, b2, *, rb):
    B, H, W, Cin = x.shape
    C2 = int(w_cat.shape[-1])
    RB = min(rb, H)
    NB = H // RB
    xf = x.reshape(B, H * W, Cin)

    def _body(x_ref, w_ref, b_ref, o_ref):
        xb = x_ref[0]
        for dy in range(2):
            y = jnp.dot(xb, w_ref[dy],
                        preferred_element_type=jnp.float32) + b_ref[...]
            o_ref[0, :, dy] = y.reshape(RB, W, C2).astype(jnp.bfloat16)

    out = pl.pallas_call(
        _body,
        out_shape=jax.ShapeDtypeStruct((B, H, 2, W, C2), jnp.bfloat16),
        grid=(B, NB),
        in_specs=[
            pl.BlockSpec((1, RB * W, Cin), lambda bb, ii: (bb, ii, 0)),
            pl.BlockSpec((2, Cin, C2), lambda bb, ii: (0, 0, 0)),
            pl.BlockSpec((1, C2), lambda bb, ii: (0, 0)),
        ],
        out_specs=pl.BlockSpec((1, RB, 2, W, C2),
                               lambda bb, ii: (bb, ii, 0, 0, 0)),
        compiler_params=_params(("parallel", "parallel")),
    )(xf, w_cat, b2)
    return out.reshape(B, 2 * H, 2 * W, C2 // 2)


def kernel(x_nchw, inc_w, inc_b, d1_w, d1_b, d2_w, d2_b, up1_tw, up1_tb,
           up1_ws, up1_wu, up1_b, up2_tw, up2_tb, up2_ws, up2_wu, up2_b,
           out_w, out_b):
    f16 = jnp.bfloat16
    x = jnp.transpose(x_nchw, (0, 2, 3, 1))
    B, H, W, Cin = x.shape

    xp = jnp.pad(x, ((0, 0), (1, 1), (1, 1), (0, 0)))
    xcol = jnp.concatenate(
        [xp[:, dy:dy + H, dx:dx + W, :] for dy in range(3) for dx in range(3)],
        axis=-1).astype(f16)
    w_stem = inc_w.reshape(9 * Cin, -1).astype(f16)

    def tcat(w):
        return jnp.concatenate([w[0::2], w[1::2]], axis=-1).astype(f16)

    CP = 8
    n_cls = int(out_w.shape[-1])
    wo = jnp.pad(out_w, ((0, 0), (0, CP - n_cls))).astype(f16)
    bo = jnp.pad(out_b, ((0, 0), (0, CP - n_cls)))

    x1, x1p = _stem(xcol, w_stem, inc_b, rb=16)
    x2, x2p = _conv3x3([x1p], d1_w.astype(f16), d1_b, rb=16, pool=True)
    x3 = _conv3x3([x2p], d2_w.astype(f16), d2_b, rb=16)
    u1 = _convT(x3, tcat(up1_tw), jnp.concatenate([up1_tb, up1_tb], -1), rb=16)
    y1 = _conv3x3([x2, u1],
                  jnp.concatenate([up1_ws, up1_wu], axis=1).astype(f16),
                  up1_b, rb=16)
    u2 = _convT(y1, tcat(up2_tw), jnp.concatenate([up2_tb, up2_tb], -1), rb=16)
    lg = _conv3x3([x1, u2],
                  jnp.concatenate([up2_ws, up2_wu], axis=1).astype(f16),
                  up2_b, rb=16, head=(wo, bo))
    return jnp.transpose(lg[..., :n_cls], (0, 3, 1, 2))


# speedup vs baseline: 1.0964x; 1.0964x over previous
import jax
import jax.numpy as jnp
from jax.experimental import pallas as pl
from jax.experimental.pallas import tpu as pltpu

_VMEM_LIMIT = 64 * 1024 * 1024


def _params(dims):
    return pltpu.CompilerParams(dimension_semantics=dims,
                                vmem_limit_bytes=_VMEM_LIMIT)


def _conv3x3(xs, w, b, *, rb, pool=False, head=None):
    B, H, W, _ = xs[0].shape
    cins = tuple(int(x.shape[-1]) for x in xs)
    ctot = sum(cins)
    Cout = int(w.shape[-1])
    n = len(xs)
    RB = min(rb, H)
    assert H % RB == 0 and (not pool or RB % 2 == 0)
    NB = H // RB
    CT = Cout if Cout <= 256 else 256
    NC = Cout // CT
    Wp = ((W + 2 + 15) // 16) * 16
    PW = Wp - W
    M = RB * Wp
    FLAT = 16 + (RB + 2) * Wp + 16
    CP = int(head[0].shape[-1]) if head is not None else 0

    def _body(*refs):
        x_refs = refs[:3 * n]
        w_ref = refs[3 * n]
        b_ref = refs[3 * n + 1]
        pos = 3 * n + 2
        if head is not None:
            wo_ref, bo_ref = refs[pos], refs[pos + 1]
            pos += 2
        o_ref = refs[pos]
        p_ref = refs[pos + 1] if pool else None
        xf, xl, xr = refs[-3], refs[-2], refs[-1]

        i = pl.program_id(1)
        first = i == 0
        last = i == NB - 1

        @pl.when(pl.program_id(2) == 0)
        def _stage():
            off = 0
            for j in range(n):
                cin = cins[j]
                top_ref, mid_ref, bot_ref = x_refs[3 * j:3 * j + 3]
                lanes = slice(off, off + cin)
                zrow = jnp.zeros((W, cin), jnp.bfloat16)
                zpad = jnp.zeros((PW, cin), jnp.bfloat16)
                xf[pl.ds(0, 16), lanes] = jnp.zeros((16, cin), jnp.bfloat16)
                xf[pl.ds(16, W), lanes] = jnp.where(first, zrow, top_ref[0, 0])
                xf[pl.ds(16 + W, PW), lanes] = zpad
                for r in range(RB):
                    base = 16 + (r + 1) * Wp
                    xf[pl.ds(base, W), lanes] = mid_ref[0, r]
                    xf[pl.ds(base + W, PW), lanes] = zpad
                base = 16 + (RB + 1) * Wp
                xf[pl.ds(base, W), lanes] = jnp.where(last, zrow, bot_ref[0, 0])
                xf[pl.ds(base + W, PW), lanes] = zpad
                xf[pl.ds(16 + (RB + 2) * Wp, 16), lanes] = \
                    jnp.zeros((16, cin), jnp.bfloat16)
                off += cin
            xr[pl.ds(1, FLAT - 1), :] = xf[pl.ds(0, FLAT - 1), :]
            xl[pl.ds(0, FLAT - 1), :] = xf[pl.ds(1, FLAT - 1), :]

        srcs = (xr, xf, xl)
        acc = jnp.zeros((M, CT), jnp.float32)
        for dy in range(3):
            for dx in range(3):
                lhs = srcs[dx][pl.ds(16 + dy * Wp, M), :]
                acc = acc + jnp.dot(lhs, w_ref[dy * 3 + dx],
                                    preferred_element_type=jnp.float32)
        acc = jnp.maximum(acc + b_ref[...], 0.0)
        y = acc.reshape(RB, Wp, CT)[:, :W, :]
        if head is not None:
            yb = y.astype(jnp.bfloat16).reshape(RB * W, CT)
            lg = jnp.dot(yb, wo_ref[...],
                         preferred_element_type=jnp.float32) + bo_ref[...]
            o_ref[0] = lg.reshape(RB, W, CP)
        else:
            o_ref[0] = y.astype(jnp.bfloat16)
            if pool:
                t = jnp.max(y.reshape(RB // 2, 2, W, CT), axis=1)
                t2 = t.reshape(RB // 2, W // 2, 2 * CT)
                p_ref[0] = jnp.maximum(t2[:, :, :CT],
                                       t2[:, :, CT:]).astype(jnp.bfloat16)

    in_specs, inputs = [], []
    for x, cin in zip(xs, cins):
        in_specs += [
            pl.BlockSpec((1, 1, W, cin),
                         lambda bb, ii, cc: (bb, jnp.maximum(ii * RB - 1, 0), 0, 0)),
            pl.BlockSpec((1, RB, W, cin),
                         lambda bb, ii, cc: (bb, ii, 0, 0)),
            pl.BlockSpec((1, 1, W, cin),
                         lambda bb, ii, cc: (bb, jnp.minimum(ii * RB + RB, H - 1), 0, 0)),
        ]
        inputs += [x, x, x]
    in_specs.append(pl.BlockSpec((9, ctot, CT), lambda bb, ii, cc: (0, 0, cc)))
    inputs.append(w)
    in_specs.append(pl.BlockSpec((1, CT), lambda bb, ii, cc: (0, cc)))
    inputs.append(b)
    if head is not None:
        in_specs.append(pl.BlockSpec((CT, CP), lambda bb, ii, cc: (0, 0)))
        inputs.append(head[0])
        in_specs.append(pl.BlockSpec((1, CP), lambda bb, ii, cc: (0, 0)))
        inputs.append(head[1])

    if head is not None:
        out_shape = jax.ShapeDtypeStruct((B, H, W, CP), jnp.float32)
        out_specs = pl.BlockSpec((1, RB, W, CP), lambda bb, ii, cc: (bb, ii, 0, 0))
    elif pool:
        out_shape = (jax.ShapeDtypeStruct((B, H, W, Cout), jnp.bfloat16),
                     jax.ShapeDtypeStruct((B, H // 2, W // 2, Cout), jnp.bfloat16))
        out_specs = (pl.BlockSpec((1, RB, W, CT), lambda bb, ii, cc: (bb, ii, 0, cc)),
                     pl.BlockSpec((1, RB // 2, W // 2, CT),
                                  lambda bb, ii, cc: (bb, ii, 0, cc)))
    else:
        out_shape = jax.ShapeDtypeStruct((B, H, W, Cout), jnp.bfloat16)
        out_specs = pl.BlockSpec((1, RB, W, CT), lambda bb, ii, cc: (bb, ii, 0, cc))

    return pl.pallas_call(
        _body,
        out_shape=out_shape,
        grid=(B, NB, NC),
        in_specs=in_specs,
        out_specs=out_specs,
        scratch_shapes=[pltpu.VMEM((FLAT, ctot), jnp.bfloat16),
                        pltpu.VMEM((FLAT, ctot), jnp.bfloat16),
                        pltpu.VMEM((FLAT, ctot), jnp.bfloat16)],
        compiler_params=_params(("parallel", "parallel", "arbitrary")),
    )(*inputs)


def _stem(xcol, w, b, *, rb):
    B, H, W, K = xcol.shape
    Cout = int(w.shape[-1])
    RB = min(rb, H)
    NB = H // RB

    def _body(x_ref, w_ref, b_ref, o_ref, p_ref):
        acc = jnp.dot(x_ref[0].reshape(RB * W, K), w_ref[...],
                      preferred_element_type=jnp.float32)
        y = jnp.maximum(acc + b_ref[...], 0.0).reshape(RB, W, Cout)
        o_ref[0] = y.astype(jnp.bfloat16)
        t = jnp.max(y.reshape(RB // 2, 2, W, Cout), axis=1)
        t2 = t.reshape(RB // 2, W // 2, 2 * Cout)
        p_ref[0] = jnp.maximum(t2[:, :, :Cout],
                               t2[:, :, Cout:]).astype(jnp.bfloat16)

    return pl.pallas_call(
        _body,
        out_shape=(jax.ShapeDtypeStruct((B, H, W, Cout), jnp.bfloat16),
                   jax.ShapeDtypeStruct((B, H // 2, W // 2, Cout), jnp.bfloat16)),
        grid=(B, NB),
        in_specs=[
            pl.BlockSpec((1, RB, W, K), lambda bb, ii: (bb, ii, 0, 0)),
            pl.BlockSpec((K, Cout), lambda bb, ii: (0, 0)),
            pl.BlockSpec((1, Cout), lambda bb, ii: (0, 0)),
        ],
        out_specs=(pl.BlockSpec((1, RB, W, Cout), lambda bb, ii: (bb, ii, 0, 0)),
                   pl.BlockSpec((1, RB // 2, W // 2, Cout),
                                lambda bb, ii: (bb, ii, 0, 0))),
        compiler_params=_params(("parallel", "parallel")),
    )(xcol, w, b)


def _convT(x, w_cat, b2, *, rb):
    B, H, W, Cin = x.shape
    C2 = int(w_cat.shape[-1])
    RB = min(rb, H)
    NB = H // RB
    xf = x.reshape(B, H * W, Cin)

    def _body(x_ref, w_ref, b_ref, o_ref):
        xb = x_ref[0]
        for dy in range(2):
            y = jnp.dot(xb, w_ref[dy],
                        preferred_element_type=jnp.float32) + b_ref[...]
            o_ref[0, :, dy] = y.reshape(RB, W, C2).astype(jnp.bfloat16)

    out = pl.pallas_call(
        _body,
        out_shape=jax.ShapeDtypeStruct((B, H, 2, W, C2), jnp.bfloat16),
        grid=(B, NB),
        in_specs=[
            pl.BlockSpec((1, RB * W, Cin), lambda bb, ii: (bb, ii, 0)),
            pl.BlockSpec((2, Cin, C2), lambda bb, ii: (0, 0, 0)),
            pl.BlockSpec((1, C2), lambda bb, ii: (0, 0)),
        ],
        out_specs=pl.BlockSpec((1, RB, 2, W, C2),
                               lambda bb, ii: (bb, ii, 0, 0, 0)),
        compiler_params=_params(("parallel", "parallel")),
    )(xf, w_cat, b2)
    return out.reshape(B, 2 * H, 2 * W, C2 // 2)


def kernel(x_nchw, inc_w, inc_b, d1_w, d1_b, d2_w, d2_b, up1_tw, up1_tb,
           up1_ws, up1_wu, up1_b, up2_tw, up2_tb, up2_ws, up2_wu, up2_b,
           out_w, out_b):
    f16 = jnp.bfloat16
    x = jnp.transpose(x_nchw, (0, 2, 3, 1))
    B, H, W, Cin = x.shape

    xp = jnp.pad(x, ((0, 0), (1, 1), (1, 1), (0, 0)))
    xcol = jnp.concatenate(
        [xp[:, dy:dy + H, dx:dx + W, :] for dy in range(3) for dx in range(3)],
        axis=-1).astype(f16)
    w_stem = inc_w.reshape(9 * Cin, -1).astype(f16)

    def tcat(w):
        return jnp.concatenate([w[0::2], w[1::2]], axis=-1).astype(f16)

    CP = 8
    n_cls = int(out_w.shape[-1])
    wo = jnp.pad(out_w, ((0, 0), (0, CP - n_cls))).astype(f16)
    bo = jnp.pad(out_b, ((0, 0), (0, CP - n_cls)))

    x1, x1p = _stem(xcol, w_stem, inc_b, rb=32)
    x2, x2p = _conv3x3([x1p], d1_w.astype(f16), d1_b, rb=16, pool=True)
    x3 = _conv3x3([x2p], d2_w.astype(f16), d2_b, rb=16)
    u1 = _convT(x3, tcat(up1_tw), jnp.concatenate([up1_tb, up1_tb], -1), rb=16)
    y1 = _conv3x3([x2, u1],
                  jnp.concatenate([up1_ws, up1_wu], axis=1).astype(f16),
                  up1_b, rb=16)
    u2 = _convT(y1, tcat(up2_tw), jnp.concatenate([up2_tb, up2_tb], -1), rb=16)
    lg = _conv3x3([x1, u2],
                  jnp.concatenate([up2_ws, up2_wu], axis=1).astype(f16),
                  up2_b, rb=16, head=(wo, bo))
    return jnp.transpose(lg[..., :n_cls], (0, 3, 1, 2))


# speedup vs baseline: 1.2689x; 1.1574x over previous
import jax
import jax.numpy as jnp
from jax.experimental import pallas as pl
from jax.experimental.pallas import tpu as pltpu

_VMEM_LIMIT = 64 * 1024 * 1024


def _params(dims):
    return pltpu.CompilerParams(dimension_semantics=dims,
                                vmem_limit_bytes=_VMEM_LIMIT)


def _conv3x3(xs, w, b, *, rb, pool=False, head=None, up=None):
    B, H, W, _ = xs[0].shape
    cins = tuple(int(x.shape[-1]) for x in xs)
    n = len(xs)
    if up is not None:
        xu, wc, bc = up
        Cu = int(xu.shape[-1])
        Cuo = int(wc.shape[-1]) // 2
        Wh = W // 2
        cins = cins + (Cuo,)
    ctot = sum(cins)
    Cout = int(w.shape[-1])
    RB = min(rb, H)
    assert H % RB == 0 and (not pool or RB % 2 == 0)
    assert up is None or RB % 2 == 0
    RBH = RB // 2
    NB = H // RB
    CT = Cout if Cout <= 256 else 256
    NC = Cout // CT
    Wp = ((W + 2 + 15) // 16) * 16
    PW = Wp - W
    M = RB * Wp
    FLAT = 16 + (RB + 2) * Wp + 16
    CP = int(head[0].shape[-1]) if head is not None else 0

    def _body(*refs):
        x_refs = refs[:3 * n]
        pos = 3 * n
        if up is not None:
            ut_ref, um_ref, ub_ref = refs[pos:pos + 3]
            wc_ref, bc_ref = refs[pos + 3], refs[pos + 4]
            pos += 5
        w_ref = refs[pos]
        b_ref = refs[pos + 1]
        pos += 2
        if head is not None:
            wo_ref, bo_ref = refs[pos], refs[pos + 1]
            pos += 2
        o_ref = refs[pos]
        p_ref = refs[pos + 1] if pool else None
        xf, xl, xr = refs[-3], refs[-2], refs[-1]

        i = pl.program_id(1)
        first = i == 0
        last = i == NB - 1

        @pl.when(pl.program_id(2) == 0)
        def _stage():
            off = 0
            for j in range(n if up is None else n + 1):
                cin = cins[j]
                lanes = slice(off, off + cin)
                zrow = jnp.zeros((W, cin), jnp.bfloat16)
                zpad = jnp.zeros((PW, cin), jnp.bfloat16)
                xf[pl.ds(0, 16), lanes] = jnp.zeros((16, cin), jnp.bfloat16)
                xf[pl.ds(16 + W, PW), lanes] = zpad
                for r in range(RB):
                    xf[pl.ds(16 + (r + 1) * Wp + W, PW), lanes] = zpad
                xf[pl.ds(16 + (RB + 1) * Wp + W, PW), lanes] = zpad
                xf[pl.ds(16 + (RB + 2) * Wp, 16), lanes] = \
                    jnp.zeros((16, cin), jnp.bfloat16)
                if j < n:
                    top_ref, mid_ref, bot_ref = x_refs[3 * j:3 * j + 3]
                    xf[pl.ds(16, W), lanes] = \
                        jnp.where(first, zrow, top_ref[0, 0])
                    for r in range(RB):
                        xf[pl.ds(16 + (r + 1) * Wp, W), lanes] = mid_ref[0, r]
                    xf[pl.ds(16 + (RB + 1) * Wp, W), lanes] = \
                        jnp.where(last, zrow, bot_ref[0, 0])
                else:
                    ut = (jnp.dot(ut_ref[0, 0], wc_ref[1],
                                  preferred_element_type=jnp.float32)
                          + bc_ref[...]).astype(jnp.bfloat16)
                    xf[pl.ds(16, W), lanes] = \
                        jnp.where(first, zrow, ut.reshape(W, cin))
                    ub = (jnp.dot(ub_ref[0, 0], wc_ref[0],
                                  preferred_element_type=jnp.float32)
                          + bc_ref[...]).astype(jnp.bfloat16)
                    xf[pl.ds(16 + (RB + 1) * Wp, W), lanes] = \
                        jnp.where(last, zrow, ub.reshape(W, cin))
                    xb = um_ref[0].reshape(RBH * Wh, Cu)
                    for dy in range(2):
                        ud = (jnp.dot(xb, wc_ref[dy],
                                      preferred_element_type=jnp.float32)
                              + bc_ref[...]).astype(jnp.bfloat16)
                        ud = ud.reshape(RBH, Wh, 2 * cin)
                        for r in range(RBH):
                            base = 16 + (2 * r + dy + 1) * Wp
                            xf[pl.ds(base, W), lanes] = \
                                ud[r].reshape(W, cin)
                off += cin
            xr[pl.ds(1, FLAT - 1), :] = xf[pl.ds(0, FLAT - 1), :]
            xl[pl.ds(0, FLAT - 1), :] = xf[pl.ds(1, FLAT - 1), :]

        srcs = (xr, xf, xl)
        acc = jnp.zeros((M, CT), jnp.float32)
        for dy in range(3):
            for dx in range(3):
                lhs = srcs[dx][pl.ds(16 + dy * Wp, M), :]
                acc = acc + jnp.dot(lhs, w_ref[dy * 3 + dx],
                                    preferred_element_type=jnp.float32)
        acc = jnp.maximum(acc + b_ref[...], 0.0)
        y = acc.reshape(RB, Wp, CT)[:, :W, :]
        if head is not None:
            yb = y.astype(jnp.bfloat16).reshape(RB * W, CT)
            lg = jnp.dot(yb, wo_ref[...],
                         preferred_element_type=jnp.float32) + bo_ref[...]
            o_ref[0] = lg.reshape(RB, W, CP)
        else:
            o_ref[0] = y.astype(jnp.bfloat16)
            if pool:
                t = jnp.max(y.reshape(RB // 2, 2, W, CT), axis=1)
                t2 = t.reshape(RB // 2, W // 2, 2 * CT)
                p_ref[0] = jnp.maximum(t2[:, :, :CT],
                                       t2[:, :, CT:]).astype(jnp.bfloat16)

    in_specs, inputs = [], []
    for x, cin in zip(xs, cins):
        in_specs += [
            pl.BlockSpec((1, 1, W, cin),
                         lambda bb, ii, cc: (bb, jnp.maximum(ii * RB - 1, 0), 0, 0)),
            pl.BlockSpec((1, RB, W, cin),
                         lambda bb, ii, cc: (bb, ii, 0, 0)),
            pl.BlockSpec((1, 1, W, cin),
                         lambda bb, ii, cc: (bb, jnp.minimum(ii * RB + RB, H - 1), 0, 0)),
        ]
        inputs += [x, x, x]
    if up is not None:
        HH = H // 2
        in_specs += [
            pl.BlockSpec((1, 1, Wh, Cu),
                         lambda bb, ii, cc: (bb, jnp.maximum(ii * RBH - 1, 0), 0, 0)),
            pl.BlockSpec((1, RBH, Wh, Cu),
                         lambda bb, ii, cc: (bb, ii, 0, 0)),
            pl.BlockSpec((1, 1, Wh, Cu),
                         lambda bb, ii, cc: (bb, jnp.minimum(ii * RBH + RBH, HH - 1), 0, 0)),
            pl.BlockSpec((2, Cu, 2 * Cuo), lambda bb, ii, cc: (0, 0, 0)),
            pl.BlockSpec((1, 2 * Cuo), lambda bb, ii, cc: (0, 0)),
        ]
        inputs += [xu, xu, xu, wc, bc]
    in_specs.append(pl.BlockSpec((9, ctot, CT), lambda bb, ii, cc: (0, 0, cc)))
    inputs.append(w)
    in_specs.append(pl.BlockSpec((1, CT), lambda bb, ii, cc: (0, cc)))
    inputs.append(b)
    if head is not None:
        in_specs.append(pl.BlockSpec((CT, CP), lambda bb, ii, cc: (0, 0)))
        inputs.append(head[0])
        in_specs.append(pl.BlockSpec((1, CP), lambda bb, ii, cc: (0, 0)))
        inputs.append(head[1])

    if head is not None:
        out_shape = jax.ShapeDtypeStruct((B, H, W, CP), jnp.float32)
        out_specs = pl.BlockSpec((1, RB, W, CP), lambda bb, ii, cc: (bb, ii, 0, 0))
    elif pool:
        out_shape = (jax.ShapeDtypeStruct((B, H, W, Cout), jnp.bfloat16),
                     jax.ShapeDtypeStruct((B, H // 2, W // 2, Cout), jnp.bfloat16))
        out_specs = (pl.BlockSpec((1, RB, W, CT), lambda bb, ii, cc: (bb, ii, 0, cc)),
                     pl.BlockSpec((1, RB // 2, W // 2, CT),
                                  lambda bb, ii, cc: (bb, ii, 0, cc)))
    else:
        out_shape = jax.ShapeDtypeStruct((B, H, W, Cout), jnp.bfloat16)
        out_specs = pl.BlockSpec((1, RB, W, CT), lambda bb, ii, cc: (bb, ii, 0, cc))

    return pl.pallas_call(
        _body,
        out_shape=out_shape,
        grid=(B, NB, NC),
        in_specs=in_specs,
        out_specs=out_specs,
        scratch_shapes=[pltpu.VMEM((FLAT, ctot), jnp.bfloat16),
                        pltpu.VMEM((FLAT, ctot), jnp.bfloat16),
                        pltpu.VMEM((FLAT, ctot), jnp.bfloat16)],
        compiler_params=_params(("parallel", "parallel", "arbitrary")),
    )(*inputs)


def _stem(xcol, w, b, *, rb):
    B, H, W, K = xcol.shape
    Cout = int(w.shape[-1])
    RB = min(rb, H)
    NB = H // RB

    def _body(x_ref, w_ref, b_ref, o_ref, p_ref):
        acc = jnp.dot(x_ref[0].reshape(RB * W, K), w_ref[...],
                      preferred_element_type=jnp.float32)
        y = jnp.maximum(acc + b_ref[...], 0.0).reshape(RB, W, Cout)
        o_ref[0] = y.astype(jnp.bfloat16)
        t = jnp.max(y.reshape(RB // 2, 2, W, Cout), axis=1)
        t2 = t.reshape(RB // 2, W // 2, 2 * Cout)
        p_ref[0] = jnp.maximum(t2[:, :, :Cout],
                               t2[:, :, Cout:]).astype(jnp.bfloat16)

    return pl.pallas_call(
        _body,
        out_shape=(jax.ShapeDtypeStruct((B, H, W, Cout), jnp.bfloat16),
                   jax.ShapeDtypeStruct((B, H // 2, W // 2, Cout), jnp.bfloat16)),
        grid=(B, NB),
        in_specs=[
            pl.BlockSpec((1, RB, W, K), lambda bb, ii: (bb, ii, 0, 0)),
            pl.BlockSpec((K, Cout), lambda bb, ii: (0, 0)),
            pl.BlockSpec((1, Cout), lambda bb, ii: (0, 0)),
        ],
        out_specs=(pl.BlockSpec((1, RB, W, Cout), lambda bb, ii: (bb, ii, 0, 0)),
                   pl.BlockSpec((1, RB // 2, W // 2, Cout),
                                lambda bb, ii: (bb, ii, 0, 0))),
        compiler_params=_params(("parallel", "parallel")),
    )(xcol, w, b)


def _convT(x, w_cat, b2, *, rb):
    B, H, W, Cin = x.shape
    C2 = int(w_cat.shape[-1])
    RB = min(rb, H)
    NB = H // RB
    xf = x.reshape(B, H * W, Cin)

    def _body(x_ref, w_ref, b_ref, o_ref):
        xb = x_ref[0]
        for dy in range(2):
            y = jnp.dot(xb, w_ref[dy],
                        preferred_element_type=jnp.float32) + b_ref[...]
            o_ref[0, :, dy] = y.reshape(RB, W, C2).astype(jnp.bfloat16)

    out = pl.pallas_call(
        _body,
        out_shape=jax.ShapeDtypeStruct((B, H, 2, W, C2), jnp.bfloat16),
        grid=(B, NB),
        in_specs=[
            pl.BlockSpec((1, RB * W, Cin), lambda bb, ii: (bb, ii, 0)),
            pl.BlockSpec((2, Cin, C2), lambda bb, ii: (0, 0, 0)),
            pl.BlockSpec((1, C2), lambda bb, ii: (0, 0)),
        ],
        out_specs=pl.BlockSpec((1, RB, 2, W, C2),
                               lambda bb, ii: (bb, ii, 0, 0, 0)),
        compiler_params=_params(("parallel", "parallel")),
    )(xf, w_cat, b2)
    return out.reshape(B, 2 * H, 2 * W, C2 // 2)


def kernel(x_nchw, inc_w, inc_b, d1_w, d1_b, d2_w, d2_b, up1_tw, up1_tb,
           up1_ws, up1_wu, up1_b, up2_tw, up2_tb, up2_ws, up2_wu, up2_b,
           out_w, out_b):
    f16 = jnp.bfloat16
    x = jnp.transpose(x_nchw, (0, 2, 3, 1))
    B, H, W, Cin = x.shape

    xp = jnp.pad(x, ((0, 0), (1, 1), (1, 1), (0, 0)))
    xcol = jnp.concatenate(
        [xp[:, dy:dy + H, dx:dx + W, :] for dy in range(3) for dx in range(3)],
        axis=-1).astype(f16)
    w_stem = inc_w.reshape(9 * Cin, -1).astype(f16)

    def tcat(w):
        return jnp.concatenate([w[0::2], w[1::2]], axis=-1).astype(f16)

    CP = 8
    n_cls = int(out_w.shape[-1])
    wo = jnp.pad(out_w, ((0, 0), (0, CP - n_cls))).astype(f16)
    bo = jnp.pad(out_b, ((0, 0), (0, CP - n_cls)))

    x1, x1p = _stem(xcol, w_stem, inc_b, rb=32)
    x2, x2p = _conv3x3([x1p], d1_w.astype(f16), d1_b, rb=16, pool=True)
    x3 = _conv3x3([x2p], d2_w.astype(f16), d2_b, rb=16)
    y1 = _conv3x3([x2],
                  jnp.concatenate([up1_ws, up1_wu], axis=1).astype(f16),
                  up1_b, rb=16,
                  up=(x3, tcat(up1_tw), jnp.concatenate([up1_tb, up1_tb], -1)))
    lg = _conv3x3([x1],
                  jnp.concatenate([up2_ws, up2_wu], axis=1).astype(f16),
                  up2_b, rb=16, head=(wo, bo),
                  up=(y1, tcat(up2_tw), jnp.concatenate([up2_tb, up2_tb], -1)))
    return jnp.transpose(lg[..., :n_cls], (0, 3, 1, 2))


# speedup vs baseline: 1.3007x; 1.0250x over previous
import jax
import jax.numpy as jnp
from jax.experimental import pallas as pl
from jax.experimental.pallas import tpu as pltpu

_VMEM_LIMIT = 64 * 1024 * 1024


def _params(dims):
    return pltpu.CompilerParams(dimension_semantics=dims,
                                vmem_limit_bytes=_VMEM_LIMIT)


def _conv3x3(xs, w, b, *, rb, pool=False, head=None, up=None):
    B, H, W, _ = xs[0].shape
    cins = tuple(int(x.shape[-1]) for x in xs)
    n = len(xs)
    if up is not None:
        xu, wc, bc = up
        Cu = int(xu.shape[-1])
        Cuo = int(wc.shape[-1]) // 2
        Wh = W // 2
        cins = cins + (Cuo,)
    ctot = sum(cins)
    Cout = int(w.shape[-1])
    RB = min(rb, H)
    assert H % RB == 0 and (not pool or RB % 2 == 0)
    assert up is None or RB % 2 == 0
    RBH = RB // 2
    NB = H // RB
    CT = Cout if Cout <= 256 else 256
    NC = Cout // CT
    Wp = ((W + 2 + 15) // 16) * 16
    PW = Wp - W
    M = RB * Wp
    FLAT = 16 + (RB + 2) * Wp + 16
    CP = int(head[0].shape[-1]) if head is not None else 0

    def _body(*refs):
        x_refs = refs[:3 * n]
        pos = 3 * n
        if up is not None:
            ut_ref, um_ref, ub_ref = refs[pos:pos + 3]
            wc_ref, bc_ref = refs[pos + 3], refs[pos + 4]
            pos += 5
        w_ref = refs[pos]
        b_ref = refs[pos + 1]
        pos += 2
        if head is not None:
            wo_ref, bo_ref = refs[pos], refs[pos + 1]
            pos += 2
        o_ref = refs[pos]
        p_ref = refs[pos + 1] if pool else None
        xf, xl, xr = refs[-3], refs[-2], refs[-1]

        i = pl.program_id(1)
        first = i == 0
        last = i == NB - 1

        @pl.when(pl.program_id(2) == 0)
        def _stage():
            off = 0
            for j in range(n if up is None else n + 1):
                cin = cins[j]
                lanes = slice(off, off + cin)
                zrow = jnp.zeros((W, cin), jnp.bfloat16)
                zpad = jnp.zeros((PW, cin), jnp.bfloat16)
                xf[pl.ds(0, 16), lanes] = jnp.zeros((16, cin), jnp.bfloat16)
                xf[pl.ds(16 + W, PW), lanes] = zpad
                for r in range(RB):
                    xf[pl.ds(16 + (r + 1) * Wp + W, PW), lanes] = zpad
                xf[pl.ds(16 + (RB + 1) * Wp + W, PW), lanes] = zpad
                xf[pl.ds(16 + (RB + 2) * Wp, 16), lanes] = \
                    jnp.zeros((16, cin), jnp.bfloat16)
                if j < n:
                    top_ref, mid_ref, bot_ref = x_refs[3 * j:3 * j + 3]
                    xf[pl.ds(16, W), lanes] = \
                        jnp.where(first, zrow, top_ref[0, 0])
                    for r in range(RB):
                        xf[pl.ds(16 + (r + 1) * Wp, W), lanes] = mid_ref[0, r]
                    xf[pl.ds(16 + (RB + 1) * Wp, W), lanes] = \
                        jnp.where(last, zrow, bot_ref[0, 0])
                else:
                    ut = (jnp.dot(ut_ref[0, 0], wc_ref[1],
                                  preferred_element_type=jnp.float32)
                          + bc_ref[...]).astype(jnp.bfloat16)
                    xf[pl.ds(16, W), lanes] = \
                        jnp.where(first, zrow, ut.reshape(W, cin))
                    ub = (jnp.dot(ub_ref[0, 0], wc_ref[0],
                                  preferred_element_type=jnp.float32)
                          + bc_ref[...]).astype(jnp.bfloat16)
                    xf[pl.ds(16 + (RB + 1) * Wp, W), lanes] = \
                        jnp.where(last, zrow, ub.reshape(W, cin))
                    xb = um_ref[0].reshape(RBH * Wh, Cu)
                    for dy in range(2):
                        ud = (jnp.dot(xb, wc_ref[dy],
                                      preferred_element_type=jnp.float32)
                              + bc_ref[...]).astype(jnp.bfloat16)
                        ud = ud.reshape(RBH, Wh, 2 * cin)
                        for r in range(RBH):
                            base = 16 + (2 * r + dy + 1) * Wp
                            xf[pl.ds(base, W), lanes] = \
                                ud[r].reshape(W, cin)
                off += cin
            xr[pl.ds(1, FLAT - 1), :] = xf[pl.ds(0, FLAT - 1), :]
            xl[pl.ds(0, FLAT - 1), :] = xf[pl.ds(1, FLAT - 1), :]

        srcs = (xr, xf, xl)
        acc = jnp.zeros((M, CT), jnp.float32)
        for dy in range(3):
            for dx in range(3):
                lhs = srcs[dx][pl.ds(16 + dy * Wp, M), :]
                acc = acc + jnp.dot(lhs, w_ref[dy * 3 + dx],
                                    preferred_element_type=jnp.float32)
        acc = jnp.maximum(acc + b_ref[...], 0.0)
        y = acc.reshape(RB, Wp, CT)[:, :W, :]
        if head is not None:
            yb = y.astype(jnp.bfloat16).reshape(RB * W, CT)
            lg = jnp.dot(yb, wo_ref[...],
                         preferred_element_type=jnp.float32) + bo_ref[...]
            o_ref[0] = lg.reshape(RB, W, CP)
        else:
            o_ref[0] = y.astype(jnp.bfloat16)
            if pool:
                t = jnp.max(y.reshape(RB // 2, 2, W, CT), axis=1)
                t2 = t.reshape(RB // 2, W // 2, 2 * CT)
                p_ref[0] = jnp.maximum(t2[:, :, :CT],
                                       t2[:, :, CT:]).astype(jnp.bfloat16)

    in_specs, inputs = [], []
    for x, cin in zip(xs, cins):
        in_specs += [
            pl.BlockSpec((1, 1, W, cin),
                         lambda bb, ii, cc: (bb, jnp.maximum(ii * RB - 1, 0), 0, 0)),
            pl.BlockSpec((1, RB, W, cin),
                         lambda bb, ii, cc: (bb, ii, 0, 0)),
            pl.BlockSpec((1, 1, W, cin),
                         lambda bb, ii, cc: (bb, jnp.minimum(ii * RB + RB, H - 1), 0, 0)),
        ]
        inputs += [x, x, x]
    if up is not None:
        HH = H // 2
        in_specs += [
            pl.BlockSpec((1, 1, Wh, Cu),
                         lambda bb, ii, cc: (bb, jnp.maximum(ii * RBH - 1, 0), 0, 0)),
            pl.BlockSpec((1, RBH, Wh, Cu),
                         lambda bb, ii, cc: (bb, ii, 0, 0)),
            pl.BlockSpec((1, 1, Wh, Cu),
                         lambda bb, ii, cc: (bb, jnp.minimum(ii * RBH + RBH, HH - 1), 0, 0)),
            pl.BlockSpec((2, Cu, 2 * Cuo), lambda bb, ii, cc: (0, 0, 0)),
            pl.BlockSpec((1, 2 * Cuo), lambda bb, ii, cc: (0, 0)),
        ]
        inputs += [xu, xu, xu, wc, bc]
    in_specs.append(pl.BlockSpec((9, ctot, CT), lambda bb, ii, cc: (0, 0, cc)))
    inputs.append(w)
    in_specs.append(pl.BlockSpec((1, CT), lambda bb, ii, cc: (0, cc)))
    inputs.append(b)
    if head is not None:
        in_specs.append(pl.BlockSpec((CT, CP), lambda bb, ii, cc: (0, 0)))
        inputs.append(head[0])
        in_specs.append(pl.BlockSpec((1, CP), lambda bb, ii, cc: (0, 0)))
        inputs.append(head[1])

    if head is not None:
        out_shape = jax.ShapeDtypeStruct((B, H, W, CP), jnp.float32)
        out_specs = pl.BlockSpec((1, RB, W, CP), lambda bb, ii, cc: (bb, ii, 0, 0))
    elif pool:
        out_shape = (jax.ShapeDtypeStruct((B, H, W, Cout), jnp.bfloat16),
                     jax.ShapeDtypeStruct((B, H // 2, W // 2, Cout), jnp.bfloat16))
        out_specs = (pl.BlockSpec((1, RB, W, CT), lambda bb, ii, cc: (bb, ii, 0, cc)),
                     pl.BlockSpec((1, RB // 2, W // 2, CT),
                                  lambda bb, ii, cc: (bb, ii, 0, cc)))
    else:
        out_shape = jax.ShapeDtypeStruct((B, H, W, Cout), jnp.bfloat16)
        out_specs = pl.BlockSpec((1, RB, W, CT), lambda bb, ii, cc: (bb, ii, 0, cc))

    return pl.pallas_call(
        _body,
        out_shape=out_shape,
        grid=(B, NB, NC),
        in_specs=in_specs,
        out_specs=out_specs,
        scratch_shapes=[pltpu.VMEM((FLAT, ctot), jnp.bfloat16),
                        pltpu.VMEM((FLAT, ctot), jnp.bfloat16),
                        pltpu.VMEM((FLAT, ctot), jnp.bfloat16)],
        compiler_params=_params(("parallel", "parallel", "arbitrary")),
    )(*inputs)


def _stem(xcol, w, b, *, rb):
    B, H, W, K = xcol.shape
    Cout = int(w.shape[-1])
    RB = min(rb, H)
    NB = H // RB

    def _body(x_ref, w_ref, b_ref, o_ref, p_ref):
        acc = jnp.dot(x_ref[0].reshape(RB * W, K), w_ref[...],
                      preferred_element_type=jnp.float32)
        y = jnp.maximum(acc + b_ref[...], 0.0).reshape(RB, W, Cout)
        o_ref[0] = y.astype(jnp.bfloat16)
        t = jnp.max(y.reshape(RB // 2, 2, W, Cout), axis=1)
        t2 = t.reshape(RB // 2, W // 2, 2 * Cout)
        p_ref[0] = jnp.maximum(t2[:, :, :Cout],
                               t2[:, :, Cout:]).astype(jnp.bfloat16)

    return pl.pallas_call(
        _body,
        out_shape=(jax.ShapeDtypeStruct((B, H, W, Cout), jnp.bfloat16),
                   jax.ShapeDtypeStruct((B, H // 2, W // 2, Cout), jnp.bfloat16)),
        grid=(B, NB),
        in_specs=[
            pl.BlockSpec((1, RB, W, K), lambda bb, ii: (bb, ii, 0, 0)),
            pl.BlockSpec((K, Cout), lambda bb, ii: (0, 0)),
            pl.BlockSpec((1, Cout), lambda bb, ii: (0, 0)),
        ],
        out_specs=(pl.BlockSpec((1, RB, W, Cout), lambda bb, ii: (bb, ii, 0, 0)),
                   pl.BlockSpec((1, RB // 2, W // 2, Cout),
                                lambda bb, ii: (bb, ii, 0, 0))),
        compiler_params=_params(("parallel", "parallel")),
    )(xcol, w, b)


def _convT(x, w_cat, b2, *, rb):
    B, H, W, Cin = x.shape
    C2 = int(w_cat.shape[-1])
    RB = min(rb, H)
    NB = H // RB
    xf = x.reshape(B, H * W, Cin)

    def _body(x_ref, w_ref, b_ref, o_ref):
        xb = x_ref[0]
        for dy in range(2):
            y = jnp.dot(xb, w_ref[dy],
                        preferred_element_type=jnp.float32) + b_ref[...]
            o_ref[0, :, dy] = y.reshape(RB, W, C2).astype(jnp.bfloat16)

    out = pl.pallas_call(
        _body,
        out_shape=jax.ShapeDtypeStruct((B, H, 2, W, C2), jnp.bfloat16),
        grid=(B, NB),
        in_specs=[
            pl.BlockSpec((1, RB * W, Cin), lambda bb, ii: (bb, ii, 0)),
            pl.BlockSpec((2, Cin, C2), lambda bb, ii: (0, 0, 0)),
            pl.BlockSpec((1, C2), lambda bb, ii: (0, 0)),
        ],
        out_specs=pl.BlockSpec((1, RB, 2, W, C2),
                               lambda bb, ii: (bb, ii, 0, 0, 0)),
        compiler_params=_params(("parallel", "parallel")),
    )(xf, w_cat, b2)
    return out.reshape(B, 2 * H, 2 * W, C2 // 2)


def kernel(x_nchw, inc_w, inc_b, d1_w, d1_b, d2_w, d2_b, up1_tw, up1_tb,
           up1_ws, up1_wu, up1_b, up2_tw, up2_tb, up2_ws, up2_wu, up2_b,
           out_w, out_b):
    f16 = jnp.bfloat16
    x = jnp.transpose(x_nchw, (0, 2, 3, 1))
    B, H, W, Cin = x.shape

    xp = jnp.pad(x, ((0, 0), (1, 1), (1, 1), (0, 0)))
    xcol = jnp.concatenate(
        [xp[:, dy:dy + H, dx:dx + W, :] for dy in range(3) for dx in range(3)],
        axis=-1).astype(f16)
    w_stem = inc_w.reshape(9 * Cin, -1).astype(f16)

    def tcat(w):
        return jnp.concatenate([w[0::2], w[1::2]], axis=-1).astype(f16)

    CP = 8
    n_cls = int(out_w.shape[-1])
    wo = jnp.pad(out_w, ((0, 0), (0, CP - n_cls))).astype(f16)
    bo = jnp.pad(out_b, ((0, 0), (0, CP - n_cls)))

    x1, x1p = _stem(xcol, w_stem, inc_b, rb=32)
    x2, x2p = _conv3x3([x1p], d1_w.astype(f16), d1_b, rb=32, pool=True)
    x3 = _conv3x3([x2p], d2_w.astype(f16), d2_b, rb=32)
    y1 = _conv3x3([x2],
                  jnp.concatenate([up1_ws, up1_wu], axis=1).astype(f16),
                  up1_b, rb=32,
                  up=(x3, tcat(up1_tw), jnp.concatenate([up1_tb, up1_tb], -1)))
    lg = _conv3x3([x1],
                  jnp.concatenate([up2_ws, up2_wu], axis=1).astype(f16),
                  up2_b, rb=32, head=(wo, bo),
                  up=(y1, tcat(up2_tw), jnp.concatenate([up2_tb, up2_tb], -1)))
    return jnp.transpose(lg[..., :n_cls], (0, 3, 1, 2))


# speedup vs baseline: 1.4647x; 1.1261x over previous
import jax
import jax.numpy as jnp
from jax.experimental import pallas as pl
from jax.experimental.pallas import tpu as pltpu

_VMEM_LIMIT = 64 * 1024 * 1024


def _params(dims):
    return pltpu.CompilerParams(dimension_semantics=dims,
                                vmem_limit_bytes=_VMEM_LIMIT)


def _conv3x3(xs, w, b, *, rb, pool=False, head=None, up=None):
    B, H, W, _ = xs[0].shape
    cins = tuple(int(x.shape[-1]) for x in xs)
    n = len(xs)
    if up is not None:
        xu, wc, bc = up
        Cu = int(xu.shape[-1])
        Cuo = int(wc.shape[-1]) // 2
        Wh = W // 2
        cins = cins + (Cuo,)
    ctot = sum(cins)
    Cout = int(w.shape[-1])
    RB = min(rb, H)
    assert H % RB == 0 and (not pool or RB % 2 == 0)
    assert up is None or RB % 2 == 0
    RBH = RB // 2
    NB = H // RB
    CT = Cout if Cout <= 256 else 256
    NC = Cout // CT
    Wp = ((W + 2 + 15) // 16) * 16
    PW = Wp - W
    M = RB * Wp
    FLAT = 16 + (RB + 2) * Wp + 16
    CP = int(head[0].shape[-1]) if head is not None else 0

    def _body(*refs):
        x_refs = refs[:3 * n]
        pos = 3 * n
        if up is not None:
            ut_ref, um_ref, ub_ref = refs[pos:pos + 3]
            wc_ref, bc_ref = refs[pos + 3], refs[pos + 4]
            pos += 5
        w_ref = refs[pos]
        b_ref = refs[pos + 1]
        pos += 2
        if head is not None:
            wo_ref, bo_ref = refs[pos], refs[pos + 1]
            pos += 2
        o_ref = refs[pos]
        p_ref = refs[pos + 1] if pool else None
        xf, xl, xr = refs[-3], refs[-2], refs[-1]

        i = pl.program_id(1)
        first = i == 0
        last = i == NB - 1

        @pl.when(pl.program_id(2) == 0)
        def _stage():
            off = 0
            for j in range(n if up is None else n + 1):
                cin = cins[j]
                lanes = slice(off, off + cin)
                zrow = jnp.zeros((W, cin), jnp.bfloat16)
                zpad = jnp.zeros((PW, cin), jnp.bfloat16)
                xf[pl.ds(0, 16), lanes] = jnp.zeros((16, cin), jnp.bfloat16)
                xf[pl.ds(16 + W, PW), lanes] = zpad
                for r in range(RB):
                    xf[pl.ds(16 + (r + 1) * Wp + W, PW), lanes] = zpad
                xf[pl.ds(16 + (RB + 1) * Wp + W, PW), lanes] = zpad
                xf[pl.ds(16 + (RB + 2) * Wp, 16), lanes] = \
                    jnp.zeros((16, cin), jnp.bfloat16)
                if j < n:
                    top_ref, mid_ref, bot_ref = x_refs[3 * j:3 * j + 3]
                    xf[pl.ds(16, W), lanes] = \
                        jnp.where(first, zrow, top_ref[0, 0])
                    for r in range(RB):
                        xf[pl.ds(16 + (r + 1) * Wp, W), lanes] = mid_ref[0, r]
                    xf[pl.ds(16 + (RB + 1) * Wp, W), lanes] = \
                        jnp.where(last, zrow, bot_ref[0, 0])
                else:
                    ut = (jnp.dot(ut_ref[0, 0], wc_ref[1],
                                  preferred_element_type=jnp.float32)
                          + bc_ref[...]).astype(jnp.bfloat16)
                    xf[pl.ds(16, W), lanes] = \
                        jnp.where(first, zrow, ut.reshape(W, cin))
                    ub = (jnp.dot(ub_ref[0, 0], wc_ref[0],
                                  preferred_element_type=jnp.float32)
                          + bc_ref[...]).astype(jnp.bfloat16)
                    xf[pl.ds(16 + (RB + 1) * Wp, W), lanes] = \
                        jnp.where(last, zrow, ub.reshape(W, cin))
                    xb = um_ref[0].reshape(RBH * Wh, Cu)
                    for dy in range(2):
                        ud = (jnp.dot(xb, wc_ref[dy],
                                      preferred_element_type=jnp.float32)
                              + bc_ref[...]).astype(jnp.bfloat16)
                        ud = ud.reshape(RBH, Wh, 2 * cin)
                        for r in range(RBH):
                            base = 16 + (2 * r + dy + 1) * Wp
                            xf[pl.ds(base, W), lanes] = \
                                ud[r].reshape(W, cin)
                off += cin
            xr[pl.ds(1, FLAT - 1), :] = xf[pl.ds(0, FLAT - 1), :]
            xl[pl.ds(0, FLAT - 1), :] = xf[pl.ds(1, FLAT - 1), :]

        srcs = (xr, xf, xl)
        acc = jnp.zeros((M, CT), jnp.float32)
        for dy in range(3):
            for dx in range(3):
                lhs = srcs[dx][pl.ds(16 + dy * Wp, M), :]
                acc = acc + jnp.dot(lhs, w_ref[dy * 3 + dx],
                                    preferred_element_type=jnp.float32)
        acc = jnp.maximum(acc + b_ref[...], 0.0)
        y = acc.reshape(RB, Wp, CT)[:, :W, :]
        if head is not None:
            yb = y.astype(jnp.bfloat16).reshape(RB * W, CT)
            lg = jnp.dot(yb, wo_ref[...],
                         preferred_element_type=jnp.float32) + bo_ref[...]
            o_ref[0] = jnp.transpose(lg, (1, 0)).reshape(CP, RB, W)
        else:
            o_ref[0] = y.astype(jnp.bfloat16)
            if pool:
                t = jnp.max(y.reshape(RB // 2, 2, W, CT), axis=1)
                t2 = t.reshape(RB // 2, W // 2, 2 * CT)
                p_ref[0] = jnp.maximum(t2[:, :, :CT],
                                       t2[:, :, CT:]).astype(jnp.bfloat16)

    in_specs, inputs = [], []
    for x, cin in zip(xs, cins):
        in_specs += [
            pl.BlockSpec((1, 1, W, cin),
                         lambda bb, ii, cc: (bb, jnp.maximum(ii * RB - 1, 0), 0, 0)),
            pl.BlockSpec((1, RB, W, cin),
                         lambda bb, ii, cc: (bb, ii, 0, 0)),
            pl.BlockSpec((1, 1, W, cin),
                         lambda bb, ii, cc: (bb, jnp.minimum(ii * RB + RB, H - 1), 0, 0)),
        ]
        inputs += [x, x, x]
    if up is not None:
        HH = H // 2
        in_specs += [
            pl.BlockSpec((1, 1, Wh, Cu),
                         lambda bb, ii, cc: (bb, jnp.maximum(ii * RBH - 1, 0), 0, 0)),
            pl.BlockSpec((1, RBH, Wh, Cu),
                         lambda bb, ii, cc: (bb, ii, 0, 0)),
            pl.BlockSpec((1, 1, Wh, Cu),
                         lambda bb, ii, cc: (bb, jnp.minimum(ii * RBH + RBH, HH - 1), 0, 0)),
            pl.BlockSpec((2, Cu, 2 * Cuo), lambda bb, ii, cc: (0, 0, 0)),
            pl.BlockSpec((1, 2 * Cuo), lambda bb, ii, cc: (0, 0)),
        ]
        inputs += [xu, xu, xu, wc, bc]
    in_specs.append(pl.BlockSpec((9, ctot, CT), lambda bb, ii, cc: (0, 0, cc)))
    inputs.append(w)
    in_specs.append(pl.BlockSpec((1, CT), lambda bb, ii, cc: (0, cc)))
    inputs.append(b)
    if head is not None:
        in_specs.append(pl.BlockSpec((CT, CP), lambda bb, ii, cc: (0, 0)))
        inputs.append(head[0])
        in_specs.append(pl.BlockSpec((1, CP), lambda bb, ii, cc: (0, 0)))
        inputs.append(head[1])

    if head is not None:
        out_shape = jax.ShapeDtypeStruct((B, CP, H, W), jnp.float32)
        out_specs = pl.BlockSpec((1, CP, RB, W), lambda bb, ii, cc: (bb, 0, ii, 0))
    elif pool:
        out_shape = (jax.ShapeDtypeStruct((B, H, W, Cout), jnp.bfloat16),
                     jax.ShapeDtypeStruct((B, H // 2, W // 2, Cout), jnp.bfloat16))
        out_specs = (pl.BlockSpec((1, RB, W, CT), lambda bb, ii, cc: (bb, ii, 0, cc)),
                     pl.BlockSpec((1, RB // 2, W // 2, CT),
                                  lambda bb, ii, cc: (bb, ii, 0, cc)))
    else:
        out_shape = jax.ShapeDtypeStruct((B, H, W, Cout), jnp.bfloat16)
        out_specs = pl.BlockSpec((1, RB, W, CT), lambda bb, ii, cc: (bb, ii, 0, cc))

    return pl.pallas_call(
        _body,
        out_shape=out_shape,
        grid=(B, NB, NC),
        in_specs=in_specs,
        out_specs=out_specs,
        scratch_shapes=[pltpu.VMEM((FLAT, ctot), jnp.bfloat16),
                        pltpu.VMEM((FLAT, ctot), jnp.bfloat16),
                        pltpu.VMEM((FLAT, ctot), jnp.bfloat16)],
        compiler_params=_params(("parallel", "parallel", "arbitrary")),
    )(*inputs)


def _stem(xcol, w, b, *, rb):
    B, H, W, K = xcol.shape
    Cout = int(w.shape[-1])
    RB = min(rb, H)
    NB = H // RB

    def _body(x_ref, w_ref, b_ref, o_ref, p_ref):
        acc = jnp.dot(x_ref[0].reshape(RB * W, K), w_ref[...],
                      preferred_element_type=jnp.float32)
        y = jnp.maximum(acc + b_ref[...], 0.0).reshape(RB, W, Cout)
        o_ref[0] = y.astype(jnp.bfloat16)
        t = jnp.max(y.reshape(RB // 2, 2, W, Cout), axis=1)
        t2 = t.reshape(RB // 2, W // 2, 2 * Cout)
        p_ref[0] = jnp.maximum(t2[:, :, :Cout],
                               t2[:, :, Cout:]).astype(jnp.bfloat16)

    return pl.pallas_call(
        _body,
        out_shape=(jax.ShapeDtypeStruct((B, H, W, Cout), jnp.bfloat16),
                   jax.ShapeDtypeStruct((B, H // 2, W // 2, Cout), jnp.bfloat16)),
        grid=(B, NB),
        in_specs=[
            pl.BlockSpec((1, RB, W, K), lambda bb, ii: (bb, ii, 0, 0)),
            pl.BlockSpec((K, Cout), lambda bb, ii: (0, 0)),
            pl.BlockSpec((1, Cout), lambda bb, ii: (0, 0)),
        ],
        out_specs=(pl.BlockSpec((1, RB, W, Cout), lambda bb, ii: (bb, ii, 0, 0)),
                   pl.BlockSpec((1, RB // 2, W // 2, Cout),
                                lambda bb, ii: (bb, ii, 0, 0))),
        compiler_params=_params(("parallel", "parallel")),
    )(xcol, w, b)


def _convT(x, w_cat, b2, *, rb):
    B, H, W, Cin = x.shape
    C2 = int(w_cat.shape[-1])
    RB = min(rb, H)
    NB = H // RB
    xf = x.reshape(B, H * W, Cin)

    def _body(x_ref, w_ref, b_ref, o_ref):
        xb = x_ref[0]
        for dy in range(2):
            y = jnp.dot(xb, w_ref[dy],
                        preferred_element_type=jnp.float32) + b_ref[...]
            o_ref[0, :, dy] = y.reshape(RB, W, C2).astype(jnp.bfloat16)

    out = pl.pallas_call(
        _body,
        out_shape=jax.ShapeDtypeStruct((B, H, 2, W, C2), jnp.bfloat16),
        grid=(B, NB),
        in_specs=[
            pl.BlockSpec((1, RB * W, Cin), lambda bb, ii: (bb, ii, 0)),
            pl.BlockSpec((2, Cin, C2), lambda bb, ii: (0, 0, 0)),
            pl.BlockSpec((1, C2), lambda bb, ii: (0, 0)),
        ],
        out_specs=pl.BlockSpec((1, RB, 2, W, C2),
                               lambda bb, ii: (bb, ii, 0, 0, 0)),
        compiler_params=_params(("parallel", "parallel")),
    )(xf, w_cat, b2)
    return out.reshape(B, 2 * H, 2 * W, C2 // 2)


def kernel(x_nchw, inc_w, inc_b, d1_w, d1_b, d2_w, d2_b, up1_tw, up1_tb,
           up1_ws, up1_wu, up1_b, up2_tw, up2_tb, up2_ws, up2_wu, up2_b,
           out_w, out_b):
    f16 = jnp.bfloat16
    x = jnp.transpose(x_nchw, (0, 2, 3, 1))
    B, H, W, Cin = x.shape

    xp = jnp.pad(x, ((0, 0), (1, 1), (1, 1), (0, 0)))
    xcol = jnp.concatenate(
        [xp[:, dy:dy + H, dx:dx + W, :] for dy in range(3) for dx in range(3)],
        axis=-1).astype(f16)
    w_stem = inc_w.reshape(9 * Cin, -1).astype(f16)

    def tcat(w):
        return jnp.concatenate([w[0::2], w[1::2]], axis=-1).astype(f16)

    CP = 8
    n_cls = int(out_w.shape[-1])
    wo = jnp.pad(out_w, ((0, 0), (0, CP - n_cls))).astype(f16)
    bo = jnp.pad(out_b, ((0, 0), (0, CP - n_cls)))

    x1, x1p = _stem(xcol, w_stem, inc_b, rb=32)
    x2, x2p = _conv3x3([x1p], d1_w.astype(f16), d1_b, rb=32, pool=True)
    x3 = _conv3x3([x2p], d2_w.astype(f16), d2_b, rb=32)
    y1 = _conv3x3([x2],
                  jnp.concatenate([up1_ws, up1_wu], axis=1).astype(f16),
                  up1_b, rb=32,
                  up=(x3, tcat(up1_tw), jnp.concatenate([up1_tb, up1_tb], -1)))
    lg = _conv3x3([x1],
                  jnp.concatenate([up2_ws, up2_wu], axis=1).astype(f16),
                  up2_b, rb=32, head=(wo, bo),
                  up=(y1, tcat(up2_tw), jnp.concatenate([up2_tb, up2_tb], -1)))
    return lg[:, :n_cls]


# speedup vs baseline: 1.4945x; 1.0204x over previous
import jax
import jax.numpy as jnp
from jax.experimental import pallas as pl
from jax.experimental.pallas import tpu as pltpu

_VMEM_LIMIT = 64 * 1024 * 1024


def _params(dims):
    return pltpu.CompilerParams(dimension_semantics=dims,
                                vmem_limit_bytes=_VMEM_LIMIT)


def _conv3x3(xs, w, b, *, rb, pool=False, head=None, up=None):
    B, H, W, _ = xs[0].shape
    cins = tuple(int(x.shape[-1]) for x in xs)
    n = len(xs)
    if up is not None:
        xu, wc, bc = up
        Cu = int(xu.shape[-1])
        Cuo = int(wc.shape[-1]) // 2
        Wh = W // 2
        cins = cins + (Cuo,)
    ctot = sum(cins)
    Cout = int(w.shape[-1])
    RB = min(rb, H)
    assert H % RB == 0 and (not pool or RB % 2 == 0)
    assert up is None or RB % 2 == 0
    RBH = RB // 2
    NB = H // RB
    CT = Cout if Cout <= 256 else 256
    NC = Cout // CT
    Wp = ((W + 2 + 15) // 16) * 16
    PW = Wp - W
    M = RB * Wp
    FLAT = 16 + (RB + 2) * Wp + 16
    CP = int(head[0].shape[-1]) if head is not None else 0

    def _body(*refs):
        x_refs = refs[:3 * n]
        pos = 3 * n
        if up is not None:
            ut_ref, um_ref, ub_ref = refs[pos:pos + 3]
            wc_ref, bc_ref = refs[pos + 3], refs[pos + 4]
            pos += 5
        w_ref = refs[pos]
        b_ref = refs[pos + 1]
        pos += 2
        if head is not None:
            wo_ref, bo_ref = refs[pos], refs[pos + 1]
            pos += 2
        o_ref = refs[pos]
        p_ref = refs[pos + 1] if pool else None
        xf = refs[-1]

        i = pl.program_id(1)
        first = i == 0
        last = i == NB - 1

        @pl.when(pl.program_id(2) == 0)
        def _stage():
            off = 0
            for j in range(n if up is None else n + 1):
                cin = cins[j]
                lanes = slice(off, off + cin)
                zrow = jnp.zeros((W, cin), jnp.bfloat16)
                zpad = jnp.zeros((PW, cin), jnp.bfloat16)
                xf[pl.ds(0, 16), lanes] = jnp.zeros((16, cin), jnp.bfloat16)
                xf[pl.ds(16 + W, PW), lanes] = zpad
                for r in range(RB):
                    xf[pl.ds(16 + (r + 1) * Wp + W, PW), lanes] = zpad
                xf[pl.ds(16 + (RB + 1) * Wp + W, PW), lanes] = zpad
                xf[pl.ds(16 + (RB + 2) * Wp, 16), lanes] = \
                    jnp.zeros((16, cin), jnp.bfloat16)
                if j < n:
                    top_ref, mid_ref, bot_ref = x_refs[3 * j:3 * j + 3]
                    xf[pl.ds(16, W), lanes] = \
                        jnp.where(first, zrow, top_ref[0, 0])
                    for r in range(RB):
                        xf[pl.ds(16 + (r + 1) * Wp, W), lanes] = mid_ref[0, r]
                    xf[pl.ds(16 + (RB + 1) * Wp, W), lanes] = \
                        jnp.where(last, zrow, bot_ref[0, 0])
                else:
                    ut = (jnp.dot(ut_ref[0, 0], wc_ref[1],
                                  preferred_element_type=jnp.float32)
                          + bc_ref[...]).astype(jnp.bfloat16)
                    xf[pl.ds(16, W), lanes] = \
                        jnp.where(first, zrow, ut.reshape(W, cin))
                    ub = (jnp.dot(ub_ref[0, 0], wc_ref[0],
                                  preferred_element_type=jnp.float32)
                          + bc_ref[...]).astype(jnp.bfloat16)
                    xf[pl.ds(16 + (RB + 1) * Wp, W), lanes] = \
                        jnp.where(last, zrow, ub.reshape(W, cin))
                    xb = um_ref[0].reshape(RBH * Wh, Cu)
                    for dy in range(2):
                        ud = (jnp.dot(xb, wc_ref[dy],
                                      preferred_element_type=jnp.float32)
                              + bc_ref[...]).astype(jnp.bfloat16)
                        ud = ud.reshape(RBH, Wh, 2 * cin)
                        for r in range(RBH):
                            base = 16 + (2 * r + dy + 1) * Wp
                            xf[pl.ds(base, W), lanes] = \
                                ud[r].reshape(W, cin)
                off += cin
        acc = jnp.zeros((M, CT), jnp.float32)
        for dy in range(3):
            for dx in range(3):
                lhs = xf[pl.ds(15 + dy * Wp + dx, M), :]
                acc = acc + jnp.dot(lhs, w_ref[dy * 3 + dx],
                                    preferred_element_type=jnp.float32)
        acc = jnp.maximum(acc + b_ref[...], 0.0)
        y = acc.reshape(RB, Wp, CT)[:, :W, :]
        if head is not None:
            yb = y.astype(jnp.bfloat16).reshape(RB * W, CT)
            lg = jnp.dot(yb, wo_ref[...],
                         preferred_element_type=jnp.float32) + bo_ref[...]
            o_ref[0] = jnp.transpose(lg, (1, 0)).reshape(CP, RB, W)
        else:
            o_ref[0] = y.astype(jnp.bfloat16)
            if pool:
                t = jnp.max(y.reshape(RB // 2, 2, W, CT), axis=1)
                t2 = t.reshape(RB // 2, W // 2, 2 * CT)
                p_ref[0] = jnp.maximum(t2[:, :, :CT],
                                       t2[:, :, CT:]).astype(jnp.bfloat16)

    in_specs, inputs = [], []
    for x, cin in zip(xs, cins):
        in_specs += [
            pl.BlockSpec((1, 1, W, cin),
                         lambda bb, ii, cc: (bb, jnp.maximum(ii * RB - 1, 0), 0, 0)),
            pl.BlockSpec((1, RB, W, cin),
                         lambda bb, ii, cc: (bb, ii, 0, 0)),
            pl.BlockSpec((1, 1, W, cin),
                         lambda bb, ii, cc: (bb, jnp.minimum(ii * RB + RB, H - 1), 0, 0)),
        ]
        inputs += [x, x, x]
    if up is not None:
        HH = H // 2
        in_specs += [
            pl.BlockSpec((1, 1, Wh, Cu),
                         lambda bb, ii, cc: (bb, jnp.maximum(ii * RBH - 1, 0), 0, 0)),
            pl.BlockSpec((1, RBH, Wh, Cu),
                         lambda bb, ii, cc: (bb, ii, 0, 0)),
            pl.BlockSpec((1, 1, Wh, Cu),
                         lambda bb, ii, cc: (bb, jnp.minimum(ii * RBH + RBH, HH - 1), 0, 0)),
            pl.BlockSpec((2, Cu, 2 * Cuo), lambda bb, ii, cc: (0, 0, 0)),
            pl.BlockSpec((1, 2 * Cuo), lambda bb, ii, cc: (0, 0)),
        ]
        inputs += [xu, xu, xu, wc, bc]
    in_specs.append(pl.BlockSpec((9, ctot, CT), lambda bb, ii, cc: (0, 0, cc)))
    inputs.append(w)
    in_specs.append(pl.BlockSpec((1, CT), lambda bb, ii, cc: (0, cc)))
    inputs.append(b)
    if head is not None:
        in_specs.append(pl.BlockSpec((CT, CP), lambda bb, ii, cc: (0, 0)))
        inputs.append(head[0])
        in_specs.append(pl.BlockSpec((1, CP), lambda bb, ii, cc: (0, 0)))
        inputs.append(head[1])

    if head is not None:
        out_shape = jax.ShapeDtypeStruct((B, CP, H, W), jnp.float32)
        out_specs = pl.BlockSpec((1, CP, RB, W), lambda bb, ii, cc: (bb, 0, ii, 0))
    elif pool:
        out_shape = (jax.ShapeDtypeStruct((B, H, W, Cout), jnp.bfloat16),
                     jax.ShapeDtypeStruct((B, H // 2, W // 2, Cout), jnp.bfloat16))
        out_specs = (pl.BlockSpec((1, RB, W, CT), lambda bb, ii, cc: (bb, ii, 0, cc)),
                     pl.BlockSpec((1, RB // 2, W // 2, CT),
                                  lambda bb, ii, cc: (bb, ii, 0, cc)))
    else:
        out_shape = jax.ShapeDtypeStruct((B, H, W, Cout), jnp.bfloat16)
        out_specs = pl.BlockSpec((1, RB, W, CT), lambda bb, ii, cc: (bb, ii, 0, cc))

    return pl.pallas_call(
        _body,
        out_shape=out_shape,
        grid=(B, NB, NC),
        in_specs=in_specs,
        out_specs=out_specs,
        scratch_shapes=[pltpu.VMEM((FLAT, ctot), jnp.bfloat16)],
        compiler_params=_params(("parallel", "parallel", "arbitrary")),
    )(*inputs)


def _stem(xcol, w, b, *, rb):
    B, H, W, K = xcol.shape
    Cout = int(w.shape[-1])
    RB = min(rb, H)
    NB = H // RB

    def _body(x_ref, w_ref, b_ref, o_ref, p_ref):
        acc = jnp.dot(x_ref[0].reshape(RB * W, K), w_ref[...],
                      preferred_element_type=jnp.float32)
        y = jnp.maximum(acc + b_ref[...], 0.0).reshape(RB, W, Cout)
        o_ref[0] = y.astype(jnp.bfloat16)
        t = jnp.max(y.reshape(RB // 2, 2, W, Cout), axis=1)
        t2 = t.reshape(RB // 2, W // 2, 2 * Cout)
        p_ref[0] = jnp.maximum(t2[:, :, :Cout],
                               t2[:, :, Cout:]).astype(jnp.bfloat16)

    return pl.pallas_call(
        _body,
        out_shape=(jax.ShapeDtypeStruct((B, H, W, Cout), jnp.bfloat16),
                   jax.ShapeDtypeStruct((B, H // 2, W // 2, Cout), jnp.bfloat16)),
        grid=(B, NB),
        in_specs=[
            pl.BlockSpec((1, RB, W, K), lambda bb, ii: (bb, ii, 0, 0)),
            pl.BlockSpec((K, Cout), lambda bb, ii: (0, 0)),
            pl.BlockSpec((1, Cout), lambda bb, ii: (0, 0)),
        ],
        out_specs=(pl.BlockSpec((1, RB, W, Cout), lambda bb, ii: (bb, ii, 0, 0)),
                   pl.BlockSpec((1, RB // 2, W // 2, Cout),
                                lambda bb, ii: (bb, ii, 0, 0))),
        compiler_params=_params(("parallel", "parallel")),
    )(xcol, w, b)


def _convT(x, w_cat, b2, *, rb):
    B, H, W, Cin = x.shape
    C2 = int(w_cat.shape[-1])
    RB = min(rb, H)
    NB = H // RB
    xf = x.reshape(B, H * W, Cin)

    def _body(x_ref, w_ref, b_ref, o_ref):
        xb = x_ref[0]
        for dy in range(2):
            y = jnp.dot(xb, w_ref[dy],
                        preferred_element_type=jnp.float32) + b_ref[...]
            o_ref[0, :, dy] = y.reshape(RB, W, C2).astype(jnp.bfloat16)

    out = pl.pallas_call(
        _body,
        out_shape=jax.ShapeDtypeStruct((B, H, 2, W, C2), jnp.bfloat16),
        grid=(B, NB),
        in_specs=[
            pl.BlockSpec((1, RB * W, Cin), lambda bb, ii: (bb, ii, 0)),
            pl.BlockSpec((2, Cin, C2), lambda bb, ii: (0, 0, 0)),
            pl.BlockSpec((1, C2), lambda bb, ii: (0, 0)),
        ],
        out_specs=pl.BlockSpec((1, RB, 2, W, C2),
                               lambda bb, ii: (bb, ii, 0, 0, 0)),
        compiler_params=_params(("parallel", "parallel")),
    )(xf, w_cat, b2)
    return out.reshape(B, 2 * H, 2 * W, C2 // 2)


def kernel(x_nchw, inc_w, inc_b, d1_w, d1_b, d2_w, d2_b, up1_tw, up1_tb,
           up1_ws, up1_wu, up1_b, up2_tw, up2_tb, up2_ws, up2_wu, up2_b,
           out_w, out_b):
    f16 = jnp.bfloat16
    x = jnp.transpose(x_nchw, (0, 2, 3, 1))
    B, H, W, Cin = x.shape

    xp = jnp.pad(x, ((0, 0), (1, 1), (1, 1), (0, 0)))
    xcol = jnp.concatenate(
        [xp[:, dy:dy + H, dx:dx + W, :] for dy in range(3) for dx in range(3)],
        axis=-1).astype(f16)
    w_stem = inc_w.reshape(9 * Cin, -1).astype(f16)

    def tcat(w):
        return jnp.concatenate([w[0::2], w[1::2]], axis=-1).astype(f16)

    CP = 8
    n_cls = int(out_w.shape[-1])
    wo = jnp.pad(out_w, ((0, 0), (0, CP - n_cls))).astype(f16)
    bo = jnp.pad(out_b, ((0, 0), (0, CP - n_cls)))

    x1, x1p = _stem(xcol, w_stem, inc_b, rb=32)
    x2, x2p = _conv3x3([x1p], d1_w.astype(f16), d1_b, rb=32, pool=True)
    x3 = _conv3x3([x2p], d2_w.astype(f16), d2_b, rb=32)
    y1 = _conv3x3([x2],
                  jnp.concatenate([up1_ws, up1_wu], axis=1).astype(f16),
                  up1_b, rb=32,
                  up=(x3, tcat(up1_tw), jnp.concatenate([up1_tb, up1_tb], -1)))
    lg = _conv3x3([x1],
                  jnp.concatenate([up2_ws, up2_wu], axis=1).astype(f16),
                  up2_b, rb=32, head=(wo, bo),
                  up=(y1, tcat(up2_tw), jnp.concatenate([up2_tb, up2_tb], -1)))
    return lg[:, :n_cls]
